# Initial kernel scaffold; baseline (speedup 1.0000x reference)
import jax
import jax.numpy as jnp
from jax.experimental import pallas as pl


def kernel(node_feats_0, node_feats_1, out0_0, out0_1, edge_feats_0, edge_index, params):
    raise NotImplementedError("write your pallas kernel here")


# speedup vs baseline: 32.1809x; 32.1809x over previous
import functools
import numpy as np
import jax
import jax.numpy as jnp
from jax import lax
from jax.experimental import pallas as pl
from jax.experimental.pallas import tpu as pltpu
from jax.experimental.pallas import tpu_sc as plsc

N = 50000
E = 800000
C0, C1, CE, H = 32, 16, 4, 4
QK = 80
NC, NS = 2, 16
NW = NC * NS
EP = 819200
EPT = EP // NW
CH = 128
NCH = EPT // CH
G = CH // 4
ZB = 625
RPT = N // NS
BN = 2000
BE = 2048
INV_SQRT_TOT = 1.0 / np.sqrt(20.0)

_mesh = plsc.VectorSubcoreMesh(core_axis_name="c", subcore_axis_name="s")


def _build_consts():
    lane = np.arange(16, dtype=np.int32)
    e_l = lane // 4
    h_l = lane % 4
    offs = []
    for t in range(20):
        if t < 8:
            offs.append(8 * h_l + t)
        else:
            j = t - 8
            offs.append(32 + (j % 3) * 16 + 4 * h_l + (j // 3))
    ci_a = np.concatenate([e_l, h_l] + offs).astype(np.int32)
    pats = {}
    for nm, fn in [('b0', lambda t: t // 8),
                   ('b1a', lambda t: (t % 8) // 4),
                   ('b1b', lambda t: 2 + (t % 8) // 4)]:
        t0 = np.arange(16, dtype=np.int32)
        pats[nm] = np.concatenate([fn(t0), fn(t0 + 16)]).astype(np.int32)
    return ci_a, pats


_CI_A_NP, _PATS_NP = _build_consts()
_CF16_NP = np.zeros((16,), np.float32)


def _ln(x, g, b, eps=1e-5):
    m = x.mean(axis=-1, keepdims=True)
    v = ((x - m) ** 2).mean(axis=-1, keepdims=True)
    return (x - m) / jnp.sqrt(v + eps) * g + b


def _norm0(x, g, b):
    n = jnp.sqrt(x * x + 1e-12)
    t = jax.nn.relu(_ln(n, g, b))
    return x * (t / n)


def _norm1(x1, g, b):
    n = jnp.sqrt(x1[:, 0:16] ** 2 + x1[:, 16:32] ** 2 + x1[:, 32:48] ** 2 + 1e-12)
    t = jax.nn.relu(_ln(n, g, b))
    r = t / n
    return jnp.concatenate([x1[:, 16 * m:16 * m + 16] * r for m in range(3)], axis=1)


def _mm(x, w):
    return jnp.dot(x, w, preferred_element_type=jnp.float32)


def _lin1(x1, wt):
    return jnp.concatenate([_mm(x1[:, 16 * m:16 * m + 16], wt) for m in range(3)], axis=1)


def _ebias_body(e0_ref, w0_ref, w1_ref, o0_ref, o1_ref):
    x = e0_ref[...]
    o0_ref[...] = _mm(x, w0_ref[...])
    o1_ref[...] = _mm(x, w1_ref[...])


def _ebias(e0, w0, w1):
    grid = EP // BE
    return pl.pallas_call(
        _ebias_body,
        grid=(grid,),
        in_specs=[pl.BlockSpec((BE, CE), lambda i: (i, 0)),
                  pl.BlockSpec((CE, H), lambda i: (0, 0)),
                  pl.BlockSpec((CE, H), lambda i: (0, 0))],
        out_specs=[pl.BlockSpec((BE, H), lambda i: (i, 0)),
                   pl.BlockSpec((BE, H), lambda i: (i, 0))],
        out_shape=[jax.ShapeDtypeStruct((EP, H), jnp.float32),
                   jax.ShapeDtypeStruct((EP, H), jnp.float32)],
    )(e0, w0, w1)


def _qkv_body(f0_ref, f1_ref, wq0, wk0, wv0, wq1, wk1, wv1,
              qcat, kcat, v0o, v1a, v1b):
    f0 = f0_ref[...]
    f1 = f1_ref[...]
    qcat[:, 0:32] = _mm(f0, wq0[...]) * INV_SQRT_TOT
    kcat[:, 0:32] = _mm(f0, wk0[...])
    v0o[...] = _mm(f0, wv0[...])
    z8 = jnp.zeros((f0.shape[0], 8), jnp.float32)
    for m in range(3):
        x1m = f1[:, 16 * m:16 * m + 16]
        qcat[:, 32 + 16 * m:48 + 16 * m] = _mm(x1m, wq1[...]) * INV_SQRT_TOT
        kcat[:, 32 + 16 * m:48 + 16 * m] = _mm(x1m, wk1[...])
        v1m = _mm(x1m, wv1[...])
        v1a[:, 8 * m:8 * m + 8] = v1m[:, 0:8]
        v1b[:, 8 * m:8 * m + 8] = v1m[:, 8:16]
    v1a[:, 24:32] = z8
    v1b[:, 24:32] = z8


def _qkv(f0, f1, wq0, wk0, wv0, wq1, wk1, wv1):
    grid = N // BN
    nb = lambda c: pl.BlockSpec((BN, c), lambda i: (i, 0))
    wb0 = pl.BlockSpec((C0, C0), lambda i: (0, 0))
    wb1 = pl.BlockSpec((C1, C1), lambda i: (0, 0))
    return pl.pallas_call(
        _qkv_body,
        grid=(grid,),
        in_specs=[nb(C0), nb(48), wb0, wb0, wb0, wb1, wb1, wb1],
        out_specs=[nb(QK), nb(QK), nb(C0), nb(32), nb(32)],
        out_shape=[jax.ShapeDtypeStruct((N, QK), jnp.float32),
                   jax.ShapeDtypeStruct((N, QK), jnp.float32),
                   jax.ShapeDtypeStruct((N, C0), jnp.float32),
                   jax.ShapeDtypeStruct((N, 32), jnp.float32),
                   jax.ShapeDtypeStruct((N, 32), jnp.float32)],
    )(f0, f1, wq0, wk0, wv0, wq1, wk1, wv1)


def _post_body(f0_ref, f1_ref, ss_ref, a0_ref, aa_ref, ab_ref,
               wo0, wo1, g0, b0, g1, b1, nf0, nf1):
    d4 = ss_ref[0, :, 0:4] + ss_ref[1, :, 0:4] + 1e-9
    B = d4.shape[0]
    a0 = a0_ref[0] + a0_ref[1]
    aa = aa_ref[0] + aa_ref[1]
    ab = ab_ref[0] + ab_ref[1]
    d32 = jnp.broadcast_to(d4[:, :, None], (B, 4, 8)).reshape(B, 32)
    agg0 = a0 / d32
    d8a = jnp.broadcast_to(d4[:, 0:2, None], (B, 2, 4)).reshape(B, 8)
    d8b = jnp.broadcast_to(d4[:, 2:4, None], (B, 2, 4)).reshape(B, 8)
    o0 = f0_ref[...] + _mm(agg0, wo0[...])
    f1 = f1_ref[...]
    o1s = []
    for m in range(3):
        agg1m = jnp.concatenate([aa[:, 8 * m:8 * m + 8] / d8a,
                                 ab[:, 8 * m:8 * m + 8] / d8b], axis=1)
        o1s.append(f1[:, 16 * m:16 * m + 16] + _mm(agg1m, wo1[...]))
    o1 = jnp.concatenate(o1s, axis=1)
    nf0[...] = _norm0(o0, g0[...], b0[...])
    nf1[...] = _norm1(o1, g1[...], b1[...])


def _post(f0, f1, ssum_p, a0p, aap, abp, wo0, wo1, g0, b0, g1, b1):
    grid = N // BN
    nb = lambda c: pl.BlockSpec((BN, c), lambda i: (i, 0))
    pb = lambda c: pl.BlockSpec((NC, BN, c), lambda i: (0, i, 0))
    fb = lambda a, b: pl.BlockSpec((a, b), lambda i: (0, 0))
    return pl.pallas_call(
        _post_body,
        grid=(grid,),
        in_specs=[nb(C0), nb(48), pb(16), pb(32), pb(32), pb(32),
                  fb(C0, C0), fb(C1, C1), fb(1, C0), fb(1, C0), fb(1, C1), fb(1, C1)],
        out_specs=[nb(C0), nb(48)],
        out_shape=[jax.ShapeDtypeStruct((N, C0), jnp.float32),
                   jax.ShapeDtypeStruct((N, 48), jnp.float32)],
    )(f0, f1, ssum_p, a0p, aap, abp, wo0, wo1, g0, b0, g1, b1)


def _final_body(f0_ref, f1_ref, o00_ref, o01_ref,
                n0g0, n0b0, n0g1, n0b1, n1g0, n1b0, n1g1, n1b1,
                n2g0, n2b0, n2g1, n2b1, n3g0, n3b0,
                l0w0, l0w1, l1w0, l1w1, l2w0, l2w1, l3w0,
                oo0, oo1, osc, osc0):
    a0 = f0_ref[...] + o00_ref[...]
    a1 = f1_ref[...] + o01_ref[...]
    oo0[...] = a0
    oo1[...] = a1
    x0, x1 = a0, a1
    ng = [(n0g0, n0b0, n0g1, n0b1), (n1g0, n1b0, n1g1, n1b1),
          (n2g0, n2b0, n2g1, n2b1)]
    lw = [(l0w0, l0w1), (l1w0, l1w1), (l2w0, l2w1)]
    for i in range(3):
        g0, b0, g1, b1 = ng[i]
        x0 = _norm0(x0, g0[...], b0[...])
        x1 = _norm1(x1, g1[...], b1[...])
        x0 = _mm(x0, lw[i][0][...])
        x1 = _lin1(x1, lw[i][1][...])
    x0 = _norm0(x0, n3g0[...], n3b0[...])
    sc0 = _mm(x0, l3w0[...])
    B = sc0.shape[0]
    r2 = sc0.reshape(B, 7, 2)
    nrm = jnp.sqrt((r2 * r2).sum(axis=-1, keepdims=True))
    sc = (r2 / jnp.maximum(nrm, 1e-6)).reshape(B, 14)
    osc[...] = sc
    osc0[...] = sc0


def _final(f0, f1, o00, o01, prm):
    grid = N // BN
    nb = lambda c: pl.BlockSpec((BN, c), lambda i: (i, 0))
    fb = lambda a, b: pl.BlockSpec((a, b), lambda i: (0, 0))
    args = [f0, f1, o00, o01]
    specs = [nb(C0), nb(48), nb(C0), nb(48)]
    for i in range(3):
        for d in ['0', '1']:
            for t in ['g', 'b']:
                args.append(prm['lm_n%d_%s%s' % (i, t, d)].reshape(1, -1))
                specs.append(fb(1, C0 if d == '0' else C1))
    args += [prm['lm_n3_g0'].reshape(1, -1), prm['lm_n3_b0'].reshape(1, -1)]
    specs += [fb(1, C0), fb(1, C0)]
    for i in range(3):
        args += [prm['lm_l%d_w0' % i].T, prm['lm_l%d_w1' % i].T]
        specs += [fb(C0, C0), fb(C1, C1)]
    args.append(prm['lm_l3_w0'].T)
    specs.append(fb(C0, 14))
    return pl.pallas_call(
        _final_body,
        grid=(grid,),
        in_specs=specs,
        out_specs=[nb(C0), nb(48), nb(14), nb(14)],
        out_shape=[jax.ShapeDtypeStruct((N, C0), jnp.float32),
                   jax.ShapeDtypeStruct((N, 48), jnp.float32),
                   jax.ShapeDtypeStruct((N, 14), jnp.float32),
                   jax.ShapeDtypeStruct((N, 14), jnp.float32)],
    )(*args)


@functools.partial(
    pl.kernel,
    out_type=[jax.ShapeDtypeStruct((EP * 4,), jnp.float32),
              jax.ShapeDtypeStruct((NC, N, 16), jnp.float32)],
    mesh=_mesh,
    compiler_params=pltpu.CompilerParams(use_tc_tiling_on_sc=False, needs_layout_passes=False),
    scratch_types=[
        pltpu.VMEM((CH,), jnp.int32),
        pltpu.VMEM((CH,), jnp.int32),
        pltpu.VMEM((CH, QK), jnp.float32),
        pltpu.VMEM((CH, QK), jnp.float32),
        pltpu.VMEM((CH * 4,), jnp.float32),
        pltpu.VMEM((CH * 4,), jnp.float32),
        pltpu.VMEM((CH, 16), jnp.float32),
        pltpu.VMEM((ZB, 16), jnp.float32),
        pltpu.VMEM((352,), jnp.int32),
        pltpu.VMEM((16,), jnp.float32),
        pltpu.VMEM_SHARED((N, 16), jnp.float32),
        pltpu.SemaphoreType.DMA,
        pltpu.SemaphoreType.DMA,
    ])
def _pass_a(qcat_hbm, kcat_hbm, src_hbm, dst_hbm, eb_hbm, ci_hbm, cf_hbm,
            ex_hbm, ssum_hbm,
            dstv, srcv, qrows, krows, ebv, exc, ex16, zbuf, civ, cfv, ssum_sh,
            sem1, sem2):
    c = lax.axis_index("c")
    s = lax.axis_index("s")
    wid = s * NC + c
    pltpu.sync_copy(ci_hbm, civ)
    pltpu.sync_copy(cf_hbm, cfv)

    @pl.loop(0, CH)
    def z1(i):
        ex16[i, :] = cfv[...]

    @pl.loop(0, ZB)
    def z2(i):
        zbuf[i, :] = cfv[...]

    r0 = s * RPT

    @pl.loop(0, RPT // ZB)
    def zs(i):
        pltpu.sync_copy(zbuf, ssum_sh.at[pl.ds(r0 + i * ZB, ZB), :])
    plsc.subcore_barrier()

    ebase = wid * EPT

    @pl.loop(0, NCH)
    def chunk(i):
        base = ebase + i * CH
        pltpu.sync_copy(dst_hbm.at[pl.ds(base, CH)], dstv)
        pltpu.sync_copy(src_hbm.at[pl.ds(base, CH)], srcv)
        cp1 = pltpu.async_copy(qcat_hbm.at[dstv], qrows, sem1)
        cp2 = pltpu.async_copy(kcat_hbm.at[srcv], krows, sem2)
        pltpu.sync_copy(eb_hbm.at[pl.ds(base * 4, CH * 4)], ebv)
        cp1.wait()
        cp2.wait()

        @pl.loop(0, G)
        def group(g):
            e_l = civ[pl.ds(0, 16)]
            h_l = civ[pl.ds(16, 16)]
            row = g * 4 + e_l
            off = civ[pl.ds(32, 16)]
            acc = plsc.load_gather(qrows, [row, off]) * plsc.load_gather(krows, [row, off])
            for t in range(1, 20):
                off = civ[pl.ds(32 + 16 * t, 16)]
                qv = plsc.load_gather(qrows, [row, off])
                kv = plsc.load_gather(krows, [row, off])
                acc = acc + qv * kv
            score = acc + ebv[pl.ds(g * 16, 16)]
            ex = jnp.exp(score)
            exc[pl.ds(g * 16, 16)] = ex
            plsc.store_scatter(ex16, [row, h_l], ex)
        pltpu.sync_copy(exc, ex_hbm.at[pl.ds(base * 4, CH * 4)])
        pltpu.sync_copy(ex16, ssum_sh.at[dstv], add=True)
    plsc.subcore_barrier()

    @pl.loop(0, RPT // ZB)
    def wo(i):
        pltpu.sync_copy(ssum_sh.at[pl.ds(r0 + i * ZB, ZB), :],
                        ssum_hbm.at[c, pl.ds(r0 + i * ZB, ZB), :])


@functools.partial(
    pl.kernel,
    out_type=jax.ShapeDtypeStruct((NC, N, 32), jnp.float32),
    mesh=_mesh,
    compiler_params=pltpu.CompilerParams(use_tc_tiling_on_sc=False, needs_layout_passes=False),
    scratch_types=[
        pltpu.VMEM((CH,), jnp.int32),
        pltpu.VMEM((CH,), jnp.int32),
        pltpu.VMEM((CH, 32), jnp.float32),
        pltpu.VMEM((CH * 4,), jnp.float32),
        pltpu.VMEM((CH, 32), jnp.float32),
        pltpu.VMEM((ZB, 32), jnp.float32),
        pltpu.VMEM((32,), jnp.int32),
        pltpu.VMEM((16,), jnp.float32),
        pltpu.VMEM_SHARED((N, 32), jnp.float32),
        pltpu.SemaphoreType.DMA,
    ])
def _pass_b(tab_hbm, src_hbm, dst_hbm, ex_hbm, ci_hbm, cf_hbm, agg_hbm,
            srcv, dstv, vrows, exv, msg, zbuf, civ, cfv, agg_sh, sem):
    c = lax.axis_index("c")
    s = lax.axis_index("s")
    wid = s * NC + c
    pltpu.sync_copy(ci_hbm, civ)
    pltpu.sync_copy(cf_hbm, cfv)

    @pl.loop(0, ZB)
    def z2(i):
        zbuf[i, pl.ds(0, 16)] = cfv[...]
        zbuf[i, pl.ds(16, 16)] = cfv[...]
    r0 = s * RPT

    @pl.loop(0, RPT // ZB)
    def zs(i):
        pltpu.sync_copy(zbuf, agg_sh.at[pl.ds(r0 + i * ZB, ZB), :])
    plsc.subcore_barrier()

    ebase = wid * EPT

    @pl.loop(0, NCH)
    def chunk(i):
        base = ebase + i * CH
        pltpu.sync_copy(src_hbm.at[pl.ds(base, CH)], srcv)
        pltpu.sync_copy(dst_hbm.at[pl.ds(base, CH)], dstv)
        cp = pltpu.async_copy(tab_hbm.at[srcv], vrows, sem)
        pltpu.sync_copy(ex_hbm.at[pl.ds(base * 4, CH * 4)], exv)
        cp.wait()

        @pl.loop(0, CH)
        def edge(e):
            pat0 = civ[pl.ds(0, 16)]
            pat1 = civ[pl.ds(16, 16)]
            ex0 = plsc.load_gather(exv, [e * 4 + pat0])
            ex1 = plsc.load_gather(exv, [e * 4 + pat1])
            msg[e, pl.ds(0, 16)] = vrows[e, pl.ds(0, 16)] * ex0
            msg[e, pl.ds(16, 16)] = vrows[e, pl.ds(16, 16)] * ex1
        pltpu.sync_copy(msg, agg_sh.at[dstv], add=True)
    plsc.subcore_barrier()

    @pl.loop(0, RPT // ZB)
    def wo(i):
        pltpu.sync_copy(agg_sh.at[pl.ds(r0 + i * ZB, ZB), :],
                        agg_hbm.at[c, pl.ds(r0 + i * ZB, ZB), :])


def kernel(node_feats_0, node_feats_1, out0_0, out0_1, edge_feats_0,
           edge_index, params):
    f0 = node_feats_0.reshape(N, C0)
    f1 = node_feats_1.transpose(0, 2, 1).reshape(N, 48)
    pad = EP - E
    src = jnp.concatenate([edge_index[0], jnp.zeros((pad,), jnp.int32)])
    dst = jnp.concatenate([edge_index[1], jnp.zeros((pad,), jnp.int32)])
    e0 = jnp.concatenate([edge_feats_0.reshape(E, CE),
                          jnp.zeros((pad, CE), jnp.float32)])
    eb0, eb1 = _ebias(e0, params['l0_eb'], params['l1_eb'])
    kill = jnp.zeros((EP, 1), jnp.float32).at[E:, :].set(-1e30)
    eb0 = (eb0 + kill).reshape(-1)
    eb1 = (eb1 + kill).reshape(-1)
    ebias = (eb0, eb1)
    ci_a = jnp.asarray(_CI_A_NP)
    ci_b0 = jnp.asarray(_PATS_NP['b0'])
    ci_b1a = jnp.asarray(_PATS_NP['b1a'])
    ci_b1b = jnp.asarray(_PATS_NP['b1b'])
    cf = jnp.asarray(_CF16_NP)
    for l in range(2):
        pre = 'l%d' % l
        qcat, kcat, v0, v1a, v1b = _qkv(
            f0, f1, params[pre + '_q_w0'].T, params[pre + '_k_w0'].T,
            params[pre + '_v_w0'].T, params[pre + '_q_w1'].T,
            params[pre + '_k_w1'].T, params[pre + '_v_w1'].T)
        ex, ssum_p = _pass_a(qcat, kcat, src, dst, ebias[l], ci_a, cf)
        a0p = _pass_b(v0, src, dst, ex, ci_b0, cf)
        aap = _pass_b(v1a, src, dst, ex, ci_b1a, cf)
        abp = _pass_b(v1b, src, dst, ex, ci_b1b, cf)
        f0, f1 = _post(f0, f1, ssum_p, a0p, aap, abp,
                       params[pre + '_o_w0'].T, params[pre + '_o_w1'].T,
                       params[pre + '_n_g0'].reshape(1, -1),
                       params[pre + '_n_b0'].reshape(1, -1),
                       params[pre + '_n_g1'].reshape(1, -1),
                       params[pre + '_n_b1'].reshape(1, -1))
    o00 = out0_0.reshape(N, C0)
    o01 = out0_1.transpose(0, 2, 1).reshape(N, 48)
    a0, a1, sc, sc0 = _final(f0, f1, o00, o01, params)
    return (a0.reshape(N, C0, 1),
            a1.reshape(N, 3, C1).transpose(0, 2, 1),
            sc.reshape(N, 7, 2),
            sc0.reshape(N, 7, 2))
